# baseline (speedup 1.0000x reference)
import functools

import jax
import jax.numpy as jnp
from jax import lax
from jax.experimental import pallas as pl
from jax.experimental.pallas import tpu as pltpu
from jax.experimental.pallas import tpu_sc as plsc

V, T, N, K = 16, 4096, 1024, 16
LANES = 16
NWORKERS = 32
QPW = V * N // NWORKERS
CHUNKS = QPW // LANES
PAD = 128
PTOT = PAD + T + 128


def _tec_body(q_hbm, a_hbm, len_hbm, out_hbm, q_v, p_v, len_v, out_v):
    c = lax.axis_index("c")
    s = lax.axis_index("s")
    wid = s * 2 + c
    vid = wid // 2
    half = wid % 2
    q0 = half * QPW

    pltpu.sync_copy(a_hbm.at[vid], p_v.at[pl.ds(PAD, T)])
    pltpu.sync_copy(q_hbm.at[vid, pl.ds(q0, QPW)], q_v)
    pltpu.sync_copy(len_hbm, len_v)

    lane = lax.iota(jnp.int32, LANES)
    l_vec = plsc.load_gather(len_v, [jnp.full((LANES,), vid, jnp.int32)])

    neg = jnp.full((LANES,), -1.0e9, jnp.float32)
    post = jnp.full((LANES,), 1.0e9, jnp.float32)
    p_v[pl.ds(PAD - LANES, LANES)] = neg
    plsc.store_scatter(p_v, [PAD + l_vec + lane], post)
    plsc.store_scatter(p_v, [PAD + l_vec + LANES + lane], post)

    pos_cap = l_vec + PAD

    ILP = 2

    def chunk(i, carry):
        qidx = [i * (LANES * ILP) + j * LANES + lane for j in range(ILP)]
        obase = [qi * K for qi in qidx]
        q = [plsc.load_gather(q_v, [qidx[j]]) for j in range(ILP)]
        qq = [q[j] * q[j] for j in range(ILP)]
        q2 = [q[j] + q[j] for j in range(ILP)]

        def dist(j, a):
            return (qq[j] + a * a) - q2[j] * a

        pos = [jnp.full((LANES,), PAD, jnp.int32) for _ in range(ILP)]
        b = T // 2
        while b >= 1:
            for j in range(ILP):
                av = plsc.load_gather(p_v, [pos[j] + (b - 1)])
                pos[j] = jnp.where(av < q[j], pos[j] + b, pos[j])
            b //= 2
        pos = [jnp.minimum(p, pos_cap) for p in pos]

        lp = [p - 1 for p in pos]
        rp = list(pos)
        dl = [dist(j, plsc.load_gather(p_v, [lp[j]])) for j in range(ILP)]
        dr = [dist(j, plsc.load_gather(p_v, [rp[j]])) for j in range(ILP)]
        for st in range(K):
            for j in range(ILP):
                tl = dl[j] <= dr[j]
                sel = jnp.where(tl, lp[j], rp[j])
                plsc.store_scatter(out_v, [obase[j] + st], sel - PAD)
                if st == K - 1:
                    continue
                adv = jnp.where(tl, sel - 1, sel + 1)
                lp[j] = jnp.where(tl, adv, lp[j])
                rp[j] = jnp.where(tl, rp[j], adv)
                nd = dist(j, plsc.load_gather(p_v, [adv]))
                dl[j] = jnp.where(tl, nd, dl[j])
                dr[j] = jnp.where(tl, dr[j], nd)
        return carry

    plsc.parallel_loop(0, CHUNKS // ILP, unroll=2, carry=jnp.int32(0))(
        lambda i, c: chunk(i, c))
    pltpu.sync_copy(out_v, out_hbm.at[vid, pl.ds(q0 * K, QPW * K)])


def kernel(query_t, vid_timestamps, time_masks, k):
    q = query_t[..., 0]
    a = vid_timestamps[..., 0]
    lengths = jnp.sum(time_masks[:, :, 0], axis=1, dtype=jnp.int32)

    mesh = plsc.VectorSubcoreMesh(core_axis_name="c", subcore_axis_name="s")
    fn = pl.kernel(
        _tec_body,
        out_type=jax.ShapeDtypeStruct((V, N * K), jnp.int32),
        mesh=mesh,
        compiler_params=pltpu.CompilerParams(needs_layout_passes=False),
        scratch_types=[
            pltpu.VMEM((QPW,), jnp.float32),
            pltpu.VMEM((PTOT,), jnp.float32),
            pltpu.VMEM((V,), jnp.int32),
            pltpu.VMEM((QPW * K,), jnp.int32),
        ],
    )
    return fn(q, a, lengths).reshape(V, N, K)


# speedup vs baseline: 1.0052x; 1.0052x over previous
import functools

import jax
import jax.numpy as jnp
from jax import lax
from jax.experimental import pallas as pl
from jax.experimental.pallas import tpu as pltpu
from jax.experimental.pallas import tpu_sc as plsc

V, T, N, K = 16, 4096, 1024, 16
LANES = 16
NWORKERS = 32
QPW = V * N // NWORKERS
CHUNKS = QPW // LANES
PAD = 128
PTOT = PAD + T + 128


def _tec_body(q_hbm, a_hbm, len_hbm, out_hbm, q_v, p_v, len_v, out_v):
    c = lax.axis_index("c")
    s = lax.axis_index("s")
    wid = s * 2 + c
    vid = wid // 2
    half = wid % 2
    q0 = half * QPW

    pltpu.sync_copy(a_hbm.at[vid], p_v.at[pl.ds(PAD, T)])
    pltpu.sync_copy(q_hbm.at[vid, pl.ds(q0, QPW)], q_v)
    pltpu.sync_copy(len_hbm, len_v)

    lane = lax.iota(jnp.int32, LANES)
    l_vec = plsc.load_gather(len_v, [jnp.full((LANES,), vid, jnp.int32)])

    neg = jnp.full((LANES,), -1.0e9, jnp.float32)
    post = jnp.full((LANES,), 1.0e9, jnp.float32)
    p_v[pl.ds(PAD - LANES, LANES)] = neg
    plsc.store_scatter(p_v, [PAD + l_vec + lane], post)
    plsc.store_scatter(p_v, [PAD + l_vec + LANES + lane], post)

    pos_cap = l_vec + PAD

    ILP = 2

    def chunk(i, carry):
        qidx = [i * (LANES * ILP) + j * LANES + lane for j in range(ILP)]
        obase = [qi * K for qi in qidx]
        q = [plsc.load_gather(q_v, [qidx[j]]) for j in range(ILP)]
        qq = [q[j] * q[j] for j in range(ILP)]
        q2 = [q[j] + q[j] for j in range(ILP)]

        def dist(j, a):
            return (qq[j] + a * a) - q2[j] * a

        pos = [jnp.full((LANES,), PAD, jnp.int32) for _ in range(ILP)]
        b = T // 2
        while b >= 1:
            for j in range(ILP):
                av = plsc.load_gather(p_v, [pos[j] + (b - 1)])
                pos[j] = jnp.where(av < q[j], pos[j] + b, pos[j])
            b //= 2
        pos = [jnp.minimum(p, pos_cap) for p in pos]

        lp = [p - 1 for p in pos]
        rp = list(pos)
        dl = [dist(j, plsc.load_gather(p_v, [lp[j]])) for j in range(ILP)]
        dr = [dist(j, plsc.load_gather(p_v, [rp[j]])) for j in range(ILP)]
        for st in range(K):
            for j in range(ILP):
                tl = dl[j] <= dr[j]
                sel = jnp.where(tl, lp[j], rp[j])
                plsc.store_scatter(out_v, [obase[j] + st], sel - PAD)
                if st == K - 1:
                    continue
                adv = jnp.where(tl, sel - 1, sel + 1)
                lp[j] = jnp.where(tl, adv, lp[j])
                rp[j] = jnp.where(tl, rp[j], adv)
                nd = dist(j, plsc.load_gather(p_v, [adv]))
                dl[j] = jnp.where(tl, nd, dl[j])
                dr[j] = jnp.where(tl, dr[j], nd)
        return carry

    plsc.parallel_loop(0, CHUNKS // ILP, unroll=4, carry=jnp.int32(0))(
        lambda i, c: chunk(i, c))
    pltpu.sync_copy(out_v, out_hbm.at[vid, pl.ds(q0 * K, QPW * K)])


def kernel(query_t, vid_timestamps, time_masks, k):
    q = query_t[..., 0]
    a = vid_timestamps[..., 0]
    lengths = jnp.sum(time_masks[:, :, 0], axis=1, dtype=jnp.int32)

    mesh = plsc.VectorSubcoreMesh(core_axis_name="c", subcore_axis_name="s")
    fn = pl.kernel(
        _tec_body,
        out_type=jax.ShapeDtypeStruct((V, N * K), jnp.int32),
        mesh=mesh,
        compiler_params=pltpu.CompilerParams(needs_layout_passes=False),
        scratch_types=[
            pltpu.VMEM((QPW,), jnp.float32),
            pltpu.VMEM((PTOT,), jnp.float32),
            pltpu.VMEM((V,), jnp.int32),
            pltpu.VMEM((QPW * K,), jnp.int32),
        ],
    )
    return fn(q, a, lengths).reshape(V, N, K)


# speedup vs baseline: 1.0392x; 1.0338x over previous
import functools

import jax
import jax.numpy as jnp
from jax import lax
from jax.experimental import pallas as pl
from jax.experimental.pallas import tpu as pltpu
from jax.experimental.pallas import tpu_sc as plsc

V, T, N, K = 16, 4096, 1024, 16
LANES = 16
NWORKERS = 32
QPW = V * N // NWORKERS
CHUNKS = QPW // LANES
PAD = 128
PTOT = PAD + T + 128


def _tec_body(q_hbm, a_hbm, len_hbm, out_hbm, q_v, p_v, len_v, out_v):
    c = lax.axis_index("c")
    s = lax.axis_index("s")
    wid = s * 2 + c
    vid = wid // 2
    half = wid % 2
    q0 = half * QPW

    pltpu.sync_copy(a_hbm.at[vid], p_v.at[pl.ds(PAD, T)])
    pltpu.sync_copy(q_hbm.at[vid, pl.ds(q0, QPW)], q_v)
    pltpu.sync_copy(len_hbm, len_v)

    lane = lax.iota(jnp.int32, LANES)
    l_vec = plsc.load_gather(len_v, [jnp.full((LANES,), vid, jnp.int32)])

    neg = jnp.full((LANES,), -1.0e9, jnp.float32)
    post = jnp.full((LANES,), 1.0e9, jnp.float32)
    p_v[pl.ds(PAD - LANES, LANES)] = neg
    plsc.store_scatter(p_v, [PAD + l_vec + lane], post)
    plsc.store_scatter(p_v, [PAD + l_vec + LANES + lane], post)

    pos_cap = l_vec + PAD

    ILP = 2

    def chunk(i, carry):
        qidx = [i * (LANES * ILP) + j * LANES + lane for j in range(ILP)]
        q = [plsc.load_gather(q_v, [qidx[j]]) for j in range(ILP)]
        qq = [q[j] * q[j] for j in range(ILP)]
        q2 = [q[j] + q[j] for j in range(ILP)]

        def dist(j, a):
            return (qq[j] + a * a) - q2[j] * a

        pos = [jnp.full((LANES,), PAD, jnp.int32) for _ in range(ILP)]
        b = T // 2
        while b >= 1:
            for j in range(ILP):
                av = plsc.load_gather(p_v, [pos[j] + (b - 1)])
                pos[j] = jnp.where(av < q[j], pos[j] + b, pos[j])
            b //= 2
        pos = [jnp.minimum(p, pos_cap) for p in pos]

        lp = [p - 1 for p in pos]
        rp = list(pos)
        dl = [dist(j, plsc.load_gather(p_v, [lp[j]])) for j in range(ILP)]
        dr = [dist(j, plsc.load_gather(p_v, [rp[j]])) for j in range(ILP)]
        for st in range(K):
            for j in range(ILP):
                tl = dl[j] <= dr[j]
                sel = jnp.where(tl, lp[j], rp[j])
                plsc.store_scatter(
                    out_v, [qidx[j], jnp.full((LANES,), st, jnp.int32)],
                    sel - PAD)
                if st == K - 1:
                    continue
                adv = jnp.where(tl, sel - 1, sel + 1)
                lp[j] = jnp.where(tl, adv, lp[j])
                rp[j] = jnp.where(tl, rp[j], adv)
                nd = dist(j, plsc.load_gather(p_v, [adv]))
                dl[j] = jnp.where(tl, nd, dl[j])
                dr[j] = jnp.where(tl, dr[j], nd)
        return carry

    plsc.parallel_loop(0, CHUNKS // ILP, unroll=4, carry=jnp.int32(0))(
        lambda i, c: chunk(i, c))
    pltpu.sync_copy(out_v, out_hbm.at[vid, pl.ds(q0, QPW)])


def kernel(query_t, vid_timestamps, time_masks, k):
    q = query_t[..., 0]
    a = vid_timestamps[..., 0]
    lengths = jnp.sum(time_masks[:, :, 0], axis=1, dtype=jnp.int32)

    mesh = plsc.VectorSubcoreMesh(core_axis_name="c", subcore_axis_name="s")
    fn = pl.kernel(
        _tec_body,
        out_type=jax.ShapeDtypeStruct((V, N, K), jnp.int32),
        mesh=mesh,
        compiler_params=pltpu.CompilerParams(needs_layout_passes=False),
        scratch_types=[
            pltpu.VMEM((QPW,), jnp.float32),
            pltpu.VMEM((PTOT,), jnp.float32),
            pltpu.VMEM((V,), jnp.int32),
            pltpu.VMEM((QPW, K), jnp.int32),
        ],
    )
    return fn(q, a, lengths)


# speedup vs baseline: 1.0459x; 1.0065x over previous
import functools

import jax
import jax.numpy as jnp
from jax import lax
from jax.experimental import pallas as pl
from jax.experimental.pallas import tpu as pltpu
from jax.experimental.pallas import tpu_sc as plsc

V, T, N, K = 16, 4096, 1024, 16
LANES = 16
NWORKERS = 32
QPW = V * N // NWORKERS
CHUNKS = QPW // LANES
PAD = 128
PTOT = PAD + T + 128


def _tec_body(q_hbm, a_hbm, len_hbm, out_hbm, q_v, p_v, len_v, out_v):
    c = lax.axis_index("c")
    s = lax.axis_index("s")
    wid = s * 2 + c
    vid = wid // 2
    half = wid % 2
    q0 = half * QPW

    pltpu.sync_copy(a_hbm.at[vid], p_v.at[pl.ds(PAD, T)])
    pltpu.sync_copy(q_hbm.at[vid, pl.ds(q0, QPW)], q_v)
    pltpu.sync_copy(len_hbm, len_v)

    lane = lax.iota(jnp.int32, LANES)
    l_vec = plsc.load_gather(len_v, [jnp.full((LANES,), vid, jnp.int32)])

    neg = jnp.full((LANES,), -1.0e9, jnp.float32)
    post = jnp.full((LANES,), 1.0e9, jnp.float32)
    p_v[pl.ds(PAD - LANES, LANES)] = neg
    plsc.store_scatter(p_v, [PAD + l_vec + lane], post)
    plsc.store_scatter(p_v, [PAD + l_vec + LANES + lane], post)

    pos_cap = l_vec + PAD

    ILP = 1

    def chunk(i, carry):
        qidx = [i * (LANES * ILP) + j * LANES + lane for j in range(ILP)]
        q = [plsc.load_gather(q_v, [qidx[j]]) for j in range(ILP)]
        qq = [q[j] * q[j] for j in range(ILP)]
        q2 = [q[j] + q[j] for j in range(ILP)]

        def dist(j, a):
            return (qq[j] + a * a) - q2[j] * a

        pos = [jnp.full((LANES,), PAD, jnp.int32) for _ in range(ILP)]
        b = T // 2
        while b >= 1:
            for j in range(ILP):
                av = plsc.load_gather(p_v, [pos[j] + (b - 1)])
                pos[j] = jnp.where(av < q[j], pos[j] + b, pos[j])
            b //= 2
        pos = [jnp.minimum(p, pos_cap) for p in pos]

        lp = [p - 1 for p in pos]
        rp = list(pos)
        dl = [dist(j, plsc.load_gather(p_v, [lp[j]])) for j in range(ILP)]
        dr = [dist(j, plsc.load_gather(p_v, [rp[j]])) for j in range(ILP)]
        for st in range(K):
            for j in range(ILP):
                tl = dl[j] <= dr[j]
                sel = jnp.where(tl, lp[j], rp[j])
                plsc.store_scatter(
                    out_v, [qidx[j], jnp.full((LANES,), st, jnp.int32)],
                    sel - PAD)
                if st == K - 1:
                    continue
                adv = jnp.where(tl, sel - 1, sel + 1)
                lp[j] = jnp.where(tl, adv, lp[j])
                rp[j] = jnp.where(tl, rp[j], adv)
                nd = dist(j, plsc.load_gather(p_v, [adv]))
                dl[j] = jnp.where(tl, nd, dl[j])
                dr[j] = jnp.where(tl, dr[j], nd)
        return carry

    plsc.parallel_loop(0, CHUNKS // ILP, unroll=8, carry=jnp.int32(0))(
        lambda i, c: chunk(i, c))
    pltpu.sync_copy(out_v, out_hbm.at[vid, pl.ds(q0, QPW)])


def kernel(query_t, vid_timestamps, time_masks, k):
    q = query_t[..., 0]
    a = vid_timestamps[..., 0]
    lengths = jnp.sum(time_masks[:, :, 0], axis=1, dtype=jnp.int32)

    mesh = plsc.VectorSubcoreMesh(core_axis_name="c", subcore_axis_name="s")
    fn = pl.kernel(
        _tec_body,
        out_type=jax.ShapeDtypeStruct((V, N, K), jnp.int32),
        mesh=mesh,
        compiler_params=pltpu.CompilerParams(needs_layout_passes=False),
        scratch_types=[
            pltpu.VMEM((QPW,), jnp.float32),
            pltpu.VMEM((PTOT,), jnp.float32),
            pltpu.VMEM((V,), jnp.int32),
            pltpu.VMEM((QPW, K), jnp.int32),
        ],
    )
    return fn(q, a, lengths)
